# baseline (speedup 1.0000x reference)
import functools
import jax
import jax.numpy as jnp
from jax import lax
from jax.experimental import pallas as pl
from jax.experimental.pallas import tpu as pltpu
from jax.experimental.pallas import tpu_sc as plsc

_B, _N, _D = 1024, 512, 64
_T = 2 * _N
_NW = 32
_RPW = _B // _NW
_DH = _D // 2
_L = 16


def _sc_body(f_hbm, base_hbm, wn_hbm, wr_hbm, out_hbm,
             base_v, wn_v, wr_v, f_v, out_v):
    wid = lax.axis_index("s") * 2 + lax.axis_index("c")
    pltpu.sync_copy(base_hbm, base_v)
    pltpu.sync_copy(wn_hbm, wn_v)
    pltpu.sync_copy(wr_hbm, wr_v)
    row0 = wid * _RPW

    def per_row(r, carry):
        b = row0 + r
        pltpu.sync_copy(f_hbm.at[b], f_v)

        def per_dhalf(dh, carry2):
            def per_d(dd, carry3):
                d = dh * _DH + dd
                wn16 = wn_v[d, :]
                wr16 = wr_v[d, :]
                for j in range(_N // _L):
                    t0 = j * _L
                    out_v[dd, pl.ds(t0, _L)] = (
                        f_v[pl.ds(t0, _L)] * wn16 + base_v[d, pl.ds(t0, _L)])
                for j in range(_N // _L):
                    t0 = _N + j * _L
                    out_v[dd, pl.ds(t0, _L)] = (
                        f_v[pl.ds(t0, _L)] * wr16 + base_v[d, pl.ds(t0, _L)])
                return carry3

            lax.fori_loop(0, _DH, per_d, 0)
            pltpu.sync_copy(out_v, out_hbm.at[b, pl.ds(dh * _DH, _DH), :])
            return carry2

        lax.fori_loop(0, 2, per_dhalf, 0)
        return carry

    lax.fori_loop(0, _RPW, per_row, 0)


def kernel(features_nir, features_raman, W_nir, b_nir, W_raman, b_raman, pos_table, spec_table):
    B, N = features_nir.shape
    D = pos_table.shape[1]

    f_all = jnp.concatenate([features_nir, features_raman], axis=1)
    base_t = jnp.concatenate(
        [pos_table.T + (b_nir + spec_table[0])[:, None],
         pos_table.T + (b_raman + spec_table[1])[:, None]], axis=1)
    wn_splat = jnp.broadcast_to(W_nir[:, 0][:, None], (D, _L))
    wr_splat = jnp.broadcast_to(W_raman[:, 0][:, None], (D, _L))

    mesh = plsc.VectorSubcoreMesh(core_axis_name="c", subcore_axis_name="s")
    run = functools.partial(
        pl.kernel,
        out_type=jax.ShapeDtypeStruct((B, D, 2 * N), jnp.float32),
        mesh=mesh,
        scratch_types=[
            pltpu.VMEM((D, 2 * N), jnp.float32),
            pltpu.VMEM((D, _L), jnp.float32),
            pltpu.VMEM((D, _L), jnp.float32),
            pltpu.VMEM((2 * N,), jnp.float32),
            pltpu.VMEM((_DH, 2 * N), jnp.float32),
        ],
    )(_sc_body)
    out_t = run(f_all, base_t, wn_splat, wr_splat)
    return out_t.transpose(0, 2, 1)


# speedup vs baseline: 9.9593x; 9.9593x over previous
import jax
import jax.numpy as jnp
from jax.experimental import pallas as pl


def _tok_kernel(fn_ref, fr_ref, wn_ref, wr_ref, cn_ref, cr_ref, pos_ref, out_ref):
    d, n = pos_ref.shape
    tb = fn_ref.shape[0]
    base_n = pos_ref[...] + cn_ref[:, 0][:, None]
    base_r = pos_ref[...] + cr_ref[:, 0][:, None]
    f_n = jnp.broadcast_to(fn_ref[...][:, None, :], (tb, d, n))
    f_r = jnp.broadcast_to(fr_ref[...][:, None, :], (tb, d, n))
    out_ref[:, :, :n] = f_n * wn_ref[:, 0][None, :, None] + base_n[None, :, :]
    out_ref[:, :, n:] = f_r * wr_ref[:, 0][None, :, None] + base_r[None, :, :]


def kernel(features_nir, features_raman, W_nir, b_nir, W_raman, b_raman, pos_table, spec_table):
    B, N = features_nir.shape
    D = pos_table.shape[1]
    TB = 32

    pos_t = pos_table.T
    c_n = (b_nir + spec_table[0])[:, None]
    c_r = (b_raman + spec_table[1])[:, None]

    grid = (B // TB,)
    out_t = pl.pallas_call(
        _tok_kernel,
        grid=grid,
        in_specs=[
            pl.BlockSpec((TB, N), lambda i: (i, 0)),
            pl.BlockSpec((TB, N), lambda i: (i, 0)),
            pl.BlockSpec((D, 1), lambda i: (0, 0)),
            pl.BlockSpec((D, 1), lambda i: (0, 0)),
            pl.BlockSpec((D, 1), lambda i: (0, 0)),
            pl.BlockSpec((D, 1), lambda i: (0, 0)),
            pl.BlockSpec((D, N), lambda i: (0, 0)),
        ],
        out_specs=pl.BlockSpec((TB, D, 2 * N), lambda i: (i, 0, 0)),
        out_shape=jax.ShapeDtypeStruct((B, D, 2 * N), features_nir.dtype),
    )(features_nir, features_raman, W_nir, W_raman, c_n, c_r, pos_t)
    return out_t.transpose(0, 2, 1)
